# baseline (speedup 1.0000x reference)
import jax
import jax.numpy as jnp
from jax.experimental import pallas as pl
from jax.experimental.pallas import tpu as pltpu

H, M, D, KD, VD = 8, 64, 128, 128, 128
BN = 2048


def _prep_kernel(mems_ref, wk_ref, bk_ref, wv_ref, wf_ref, bv_ref,
                 mkt_ref, g_ref):
    hm2 = (H * M) // 2
    for h in range(H):
        mems_h = mems_ref[h]
        logits = jnp.dot(mems_h, wk_ref[h], preferred_element_type=jnp.float32)
        logits = logits + bk_ref[h]
        mx = jnp.max(logits, axis=1, keepdims=True)
        e = jnp.exp(logits - mx)
        mk = e / jnp.sum(e, axis=1, keepdims=True)
        mkt = mk.T
        c = h * (M // 2)
        mkt_ref[:, c:c + M // 2] = mkt[:, :M // 2]
        mkt_ref[:, hm2 + c:hm2 + c + M // 2] = mkt[:, M // 2:]
        mem_val = jnp.dot(mems_h, wv_ref[h], preferred_element_type=jnp.float32)
        mem_val = mem_val + bv_ref[h]
        gh = jnp.dot(mem_val, wf_ref[h * VD:(h + 1) * VD, :],
                     preferred_element_type=jnp.float32)
        g_ref[c:c + M // 2, :] = gh[:M // 2, :]
        g_ref[hm2 + c:hm2 + c + M // 2, :] = gh[M // 2:, :]


def _main_kernel(k_ref, mkt_ref, g_ref, seg_ref, bf_ref, o_ref):
    kb = k_ref[...]
    att = jnp.dot(kb, mkt_ref[...], preferred_element_type=jnp.float32)
    e = jnp.exp(att)
    s = jnp.dot(e, seg_ref[...], preferred_element_type=jnp.float32)
    r = 1.0 / s
    att_w = e * jnp.concatenate([r, r], axis=1)
    half = (H * M) // 2
    out = jax.lax.dot_general(
        att_w[:, :half], g_ref[:half, :],
        dimension_numbers=(((1,), (0,)), ((), ())),
        preferred_element_type=jnp.float32)
    out = out + jax.lax.dot_general(
        att_w[:, half:], g_ref[half:, :],
        dimension_numbers=(((1,), (0,)), ((), ())),
        preferred_element_type=jnp.float32)
    o_ref[...] = out + bf_ref[...]


@jax.jit
def kernel(k, mems, Wk, bk, Wv, bv, Wf, bf):
    n = k.shape[0]
    mkt, g = pl.pallas_call(
        _prep_kernel,
        out_shape=[
            jax.ShapeDtypeStruct((KD, H * M), jnp.float32),
            jax.ShapeDtypeStruct((H * M, VD), jnp.float32),
        ],
        name="mhm_prep",
    )(mems, Wk, bk.reshape(H, 1, KD), Wv, Wf, bv.reshape(H, 1, VD))

    hm = H * M
    head_of_row = (jnp.arange(hm) % (hm // 2)) // (M // 2)
    head_of_col = jnp.arange(hm // 2) // (M // 2)
    seg = (head_of_row[:, None] == head_of_col[None, :]).astype(jnp.float32)

    out = pl.pallas_call(
        _main_kernel,
        grid=(n // BN,),
        in_specs=[
            pl.BlockSpec((BN, KD), lambda i: (i, 0)),
            pl.BlockSpec((KD, hm), lambda i: (0, 0)),
            pl.BlockSpec((hm, VD), lambda i: (0, 0)),
            pl.BlockSpec((hm, hm // 2), lambda i: (0, 0)),
            pl.BlockSpec((1, VD), lambda i: (0, 0)),
        ],
        out_specs=pl.BlockSpec((BN, VD), lambda i: (i, 0)),
        out_shape=jax.ShapeDtypeStruct((n, VD), jnp.float32),
        compiler_params=pltpu.CompilerParams(
            dimension_semantics=("parallel",),
        ),
        name="mhm_main",
    )(k, mkt, g, seg, bf.reshape(1, VD))
    return out


# speedup vs baseline: 1.0645x; 1.0645x over previous
import jax
import jax.numpy as jnp
from jax.experimental import pallas as pl
from jax.experimental.pallas import tpu as pltpu

H, M, D, KD, VD = 8, 64, 128, 128, 128
BN = 4096


def _prep_kernel(mems_ref, wk_ref, bk_ref, wv_ref, wf_ref, bv_ref,
                 mkt_ref, g_ref):
    hm2 = (H * M) // 2
    for h in range(H):
        mems_h = mems_ref[h]
        logits = jnp.dot(mems_h, wk_ref[h], preferred_element_type=jnp.float32)
        logits = logits + bk_ref[h]
        mx = jnp.max(logits, axis=1, keepdims=True)
        e = jnp.exp(logits - mx)
        mk = e / jnp.sum(e, axis=1, keepdims=True)
        mkt = mk.T
        c = h * (M // 2)
        mkt_ref[:, c:c + M // 2] = mkt[:, :M // 2]
        mkt_ref[:, hm2 + c:hm2 + c + M // 2] = mkt[:, M // 2:]
        mem_val = jnp.dot(mems_h, wv_ref[h], preferred_element_type=jnp.float32)
        mem_val = mem_val + bv_ref[h]
        gh = jnp.dot(mem_val, wf_ref[h * VD:(h + 1) * VD, :],
                     preferred_element_type=jnp.float32)
        g_ref[c:c + M // 2, :] = gh[:M // 2, :]
        g_ref[hm2 + c:hm2 + c + M // 2, :] = gh[M // 2:, :]


def _main_kernel(k_ref, mkt_ref, g_ref, seg_ref, bf_ref, o_ref):
    kb = k_ref[...]
    att = jnp.dot(kb, mkt_ref[...], preferred_element_type=jnp.float32)
    e = jnp.exp(att)
    s = jnp.dot(e, seg_ref[...], preferred_element_type=jnp.float32)
    r = 1.0 / s
    att_w = e * jnp.concatenate([r, r], axis=1)
    half = (H * M) // 2
    out = jax.lax.dot_general(
        att_w[:, :half], g_ref[:half, :],
        dimension_numbers=(((1,), (0,)), ((), ())),
        preferred_element_type=jnp.float32)
    out = out + jax.lax.dot_general(
        att_w[:, half:], g_ref[half:, :],
        dimension_numbers=(((1,), (0,)), ((), ())),
        preferred_element_type=jnp.float32)
    o_ref[...] = out + bf_ref[...]


@jax.jit
def kernel(k, mems, Wk, bk, Wv, bv, Wf, bf):
    n = k.shape[0]
    mkt, g = pl.pallas_call(
        _prep_kernel,
        out_shape=[
            jax.ShapeDtypeStruct((KD, H * M), jnp.float32),
            jax.ShapeDtypeStruct((H * M, VD), jnp.float32),
        ],
        name="mhm_prep",
    )(mems, Wk, bk.reshape(H, 1, KD), Wv, Wf, bv.reshape(H, 1, VD))

    hm = H * M
    head_of_row = (jnp.arange(hm) % (hm // 2)) // (M // 2)
    head_of_col = jnp.arange(hm // 2) // (M // 2)
    seg = (head_of_row[:, None] == head_of_col[None, :]).astype(jnp.float32)

    out = pl.pallas_call(
        _main_kernel,
        grid=(n // BN,),
        in_specs=[
            pl.BlockSpec((BN, KD), lambda i: (i, 0)),
            pl.BlockSpec((KD, hm), lambda i: (0, 0)),
            pl.BlockSpec((hm, VD), lambda i: (0, 0)),
            pl.BlockSpec((hm, hm // 2), lambda i: (0, 0)),
            pl.BlockSpec((1, VD), lambda i: (0, 0)),
        ],
        out_specs=pl.BlockSpec((BN, VD), lambda i: (i, 0)),
        out_shape=jax.ShapeDtypeStruct((n, VD), jnp.float32),
        compiler_params=pltpu.CompilerParams(
            dimension_semantics=("parallel",),
        ),
        name="mhm_main",
    )(k, mkt, g, seg, bf.reshape(1, VD))
    return out


# speedup vs baseline: 1.0957x; 1.0293x over previous
import jax
import jax.numpy as jnp
from jax.experimental import pallas as pl
from jax.experimental.pallas import tpu as pltpu

H, M, D, KD, VD = 8, 64, 128, 128, 128
BN = 8192


def _prep_kernel(mems_ref, wk_ref, bk_ref, wv_ref, wf_ref, bv_ref,
                 mkt_ref, g_ref):
    hm2 = (H * M) // 2
    for h in range(H):
        mems_h = mems_ref[h]
        logits = jnp.dot(mems_h, wk_ref[h], preferred_element_type=jnp.float32)
        logits = logits + bk_ref[h]
        mx = jnp.max(logits, axis=1, keepdims=True)
        e = jnp.exp(logits - mx)
        mk = e / jnp.sum(e, axis=1, keepdims=True)
        mkt = mk.T
        c = h * (M // 2)
        mkt_ref[:, c:c + M // 2] = mkt[:, :M // 2]
        mkt_ref[:, hm2 + c:hm2 + c + M // 2] = mkt[:, M // 2:]
        mem_val = jnp.dot(mems_h, wv_ref[h], preferred_element_type=jnp.float32)
        mem_val = mem_val + bv_ref[h]
        gh = jnp.dot(mem_val, wf_ref[h * VD:(h + 1) * VD, :],
                     preferred_element_type=jnp.float32)
        g_ref[c:c + M // 2, :] = gh[:M // 2, :]
        g_ref[hm2 + c:hm2 + c + M // 2, :] = gh[M // 2:, :]


def _main_kernel(k_ref, mkt_ref, g_ref, seg_ref, bf_ref, o_ref):
    kb = k_ref[...]
    att = jnp.dot(kb, mkt_ref[...], preferred_element_type=jnp.float32)
    e = jnp.exp(att)
    s = jnp.dot(e, seg_ref[...], preferred_element_type=jnp.float32)
    r = 1.0 / s
    att_w = e * jnp.concatenate([r, r], axis=1)
    half = (H * M) // 2
    out = jax.lax.dot_general(
        att_w[:, :half], g_ref[:half, :],
        dimension_numbers=(((1,), (0,)), ((), ())),
        preferred_element_type=jnp.float32)
    out = out + jax.lax.dot_general(
        att_w[:, half:], g_ref[half:, :],
        dimension_numbers=(((1,), (0,)), ((), ())),
        preferred_element_type=jnp.float32)
    o_ref[...] = out + bf_ref[...]


@jax.jit
def kernel(k, mems, Wk, bk, Wv, bv, Wf, bf):
    n = k.shape[0]
    mkt, g = pl.pallas_call(
        _prep_kernel,
        out_shape=[
            jax.ShapeDtypeStruct((KD, H * M), jnp.float32),
            jax.ShapeDtypeStruct((H * M, VD), jnp.float32),
        ],
        name="mhm_prep",
    )(mems, Wk, bk.reshape(H, 1, KD), Wv, Wf, bv.reshape(H, 1, VD))

    hm = H * M
    head_of_row = (jnp.arange(hm) % (hm // 2)) // (M // 2)
    head_of_col = jnp.arange(hm // 2) // (M // 2)
    seg = (head_of_row[:, None] == head_of_col[None, :]).astype(jnp.float32)

    out = pl.pallas_call(
        _main_kernel,
        grid=(n // BN,),
        in_specs=[
            pl.BlockSpec((BN, KD), lambda i: (i, 0)),
            pl.BlockSpec((KD, hm), lambda i: (0, 0)),
            pl.BlockSpec((hm, VD), lambda i: (0, 0)),
            pl.BlockSpec((hm, hm // 2), lambda i: (0, 0)),
            pl.BlockSpec((1, VD), lambda i: (0, 0)),
        ],
        out_specs=pl.BlockSpec((BN, VD), lambda i: (i, 0)),
        out_shape=jax.ShapeDtypeStruct((n, VD), jnp.float32),
        compiler_params=pltpu.CompilerParams(
            dimension_semantics=("parallel",),
        ),
        name="mhm_main",
    )(k, mkt, g, seg, bf.reshape(1, VD))
    return out


# speedup vs baseline: 1.3213x; 1.2059x over previous
import jax
import jax.numpy as jnp
from jax.experimental import pallas as pl
from jax.experimental.pallas import tpu as pltpu

H, M, D, KD, VD = 8, 64, 128, 128, 128
BN = 8192


def _prep_kernel(mems_ref, wk_ref, bk_ref, wv_ref, wf_ref, bv_ref,
                 mkt_ref, g_ref):
    hm2 = (H * M) // 2
    for h in range(H):
        mems_h = mems_ref[h]
        logits = jnp.dot(mems_h, wk_ref[h], preferred_element_type=jnp.float32)
        logits = logits + bk_ref[h]
        mx = jnp.max(logits, axis=1, keepdims=True)
        e = jnp.exp(logits - mx)
        mk = e / jnp.sum(e, axis=1, keepdims=True)
        mkt = mk.T
        c = h * (M // 2)
        mkt_ref[:, c:c + M // 2] = mkt[:, :M // 2]
        mkt_ref[:, hm2 + c:hm2 + c + M // 2] = mkt[:, M // 2:]
        mem_val = jnp.dot(mems_h, wv_ref[h], preferred_element_type=jnp.float32)
        mem_val = mem_val + bv_ref[h]
        gh = jnp.dot(mem_val, wf_ref[h * VD:(h + 1) * VD, :],
                     preferred_element_type=jnp.float32)
        g_ref[c:c + M // 2, :] = gh[:M // 2, :]
        g_ref[hm2 + c:hm2 + c + M // 2, :] = gh[M // 2:, :]


CB = 2048


def _main_kernel(k_ref, mkt_ref, g_ref, seg_ref, bf_ref, o_ref):
    half = (H * M) // 2
    for c in range(BN // CB):
        rows = slice(c * CB, (c + 1) * CB)
        kb = k_ref[rows, :]
        att = jnp.dot(kb, mkt_ref[...], preferred_element_type=jnp.float32)
        e = jnp.exp(att)
        s = jnp.dot(e, seg_ref[...], preferred_element_type=jnp.float32)
        r = 1.0 / s
        att_w = e * jnp.concatenate([r, r], axis=1)
        out = jax.lax.dot_general(
            att_w[:, :half], g_ref[:half, :],
            dimension_numbers=(((1,), (0,)), ((), ())),
            preferred_element_type=jnp.float32)
        out = out + jax.lax.dot_general(
            att_w[:, half:], g_ref[half:, :],
            dimension_numbers=(((1,), (0,)), ((), ())),
            preferred_element_type=jnp.float32)
        o_ref[rows, :] = out + bf_ref[...]


@jax.jit
def kernel(k, mems, Wk, bk, Wv, bv, Wf, bf):
    n = k.shape[0]
    mkt, g = pl.pallas_call(
        _prep_kernel,
        out_shape=[
            jax.ShapeDtypeStruct((KD, H * M), jnp.float32),
            jax.ShapeDtypeStruct((H * M, VD), jnp.float32),
        ],
        name="mhm_prep",
    )(mems, Wk, bk.reshape(H, 1, KD), Wv, Wf, bv.reshape(H, 1, VD))

    hm = H * M
    head_of_row = (jnp.arange(hm) % (hm // 2)) // (M // 2)
    head_of_col = jnp.arange(hm // 2) // (M // 2)
    seg = (head_of_row[:, None] == head_of_col[None, :]).astype(jnp.float32)

    out = pl.pallas_call(
        _main_kernel,
        grid=(n // BN,),
        in_specs=[
            pl.BlockSpec((BN, KD), lambda i: (i, 0)),
            pl.BlockSpec((KD, hm), lambda i: (0, 0)),
            pl.BlockSpec((hm, VD), lambda i: (0, 0)),
            pl.BlockSpec((hm, hm // 2), lambda i: (0, 0)),
            pl.BlockSpec((1, VD), lambda i: (0, 0)),
        ],
        out_specs=pl.BlockSpec((BN, VD), lambda i: (i, 0)),
        out_shape=jax.ShapeDtypeStruct((n, VD), jnp.float32),
        compiler_params=pltpu.CompilerParams(
            dimension_semantics=("parallel",),
        ),
        name="mhm_main",
    )(k, mkt, g, seg, bf.reshape(1, VD))
    return out


# speedup vs baseline: 1.6330x; 1.2359x over previous
import jax
import jax.numpy as jnp
from jax.experimental import pallas as pl
from jax.experimental.pallas import tpu as pltpu

H, M, D, KD, VD = 8, 64, 128, 128, 128
BN = 8192


def _prep_kernel(mems_ref, wk_ref, bk_ref, wv_ref, wf_ref, bv_ref,
                 mkt_ref, g_ref):
    hm2 = (H * M) // 2
    for h in range(H):
        mems_h = mems_ref[h]
        logits = jnp.dot(mems_h, wk_ref[h], preferred_element_type=jnp.float32)
        logits = logits + bk_ref[h]
        mx = jnp.max(logits, axis=1, keepdims=True)
        e = jnp.exp(logits - mx)
        mk = e / jnp.sum(e, axis=1, keepdims=True)
        mkt = mk.T
        c = h * (M // 2)
        mkt_ref[:, c:c + M // 2] = mkt[:, :M // 2]
        mkt_ref[:, hm2 + c:hm2 + c + M // 2] = mkt[:, M // 2:]
        mem_val = jnp.dot(mems_h, wv_ref[h], preferred_element_type=jnp.float32)
        mem_val = mem_val + bv_ref[h]
        gh = jnp.dot(mem_val, wf_ref[h * VD:(h + 1) * VD, :],
                     preferred_element_type=jnp.float32)
        g_ref[c:c + M // 2, :] = gh[:M // 2, :]
        g_ref[hm2 + c:hm2 + c + M // 2, :] = gh[M // 2:, :]


CB = 2048


def _main_kernel(k_ref, mkt_ref, g_ref, seg_ref, bf_ref, o_ref):
    half = (H * M) // 2
    for c in range(BN // CB):
        rows = slice(c * CB, (c + 1) * CB)
        kb = k_ref[rows, :]
        att = jnp.dot(kb, mkt_ref[...], preferred_element_type=jnp.float32)
        e = jnp.exp(att)
        f = e[:, :half] + e[:, half:]
        s = jnp.dot(f, seg_ref[...], preferred_element_type=jnp.float32)
        r = 1.0 / s
        att_w = e * jnp.concatenate([r, r], axis=1)
        out = jax.lax.dot_general(
            att_w[:, :half], g_ref[:half, :],
            dimension_numbers=(((1,), (0,)), ((), ())),
            preferred_element_type=jnp.float32)
        out = out + jax.lax.dot_general(
            att_w[:, half:], g_ref[half:, :],
            dimension_numbers=(((1,), (0,)), ((), ())),
            preferred_element_type=jnp.float32)
        o_ref[rows, :] = out + bf_ref[...]


@jax.jit
def kernel(k, mems, Wk, bk, Wv, bv, Wf, bf):
    n = k.shape[0]
    mkt, g = pl.pallas_call(
        _prep_kernel,
        out_shape=[
            jax.ShapeDtypeStruct((KD, H * M), jnp.float32),
            jax.ShapeDtypeStruct((H * M, VD), jnp.float32),
        ],
        name="mhm_prep",
    )(mems, Wk, bk.reshape(H, 1, KD), Wv, Wf, bv.reshape(H, 1, VD))

    hm = H * M
    head_of_row = jnp.arange(hm // 2) // (M // 2)
    head_of_col = jnp.arange(hm // 2) // (M // 2)
    seg = (head_of_row[:, None] == head_of_col[None, :]).astype(jnp.float32)

    out = pl.pallas_call(
        _main_kernel,
        grid=(n // BN,),
        in_specs=[
            pl.BlockSpec((BN, KD), lambda i: (i, 0)),
            pl.BlockSpec((KD, hm), lambda i: (0, 0)),
            pl.BlockSpec((hm, VD), lambda i: (0, 0)),
            pl.BlockSpec((hm // 2, hm // 2), lambda i: (0, 0)),
            pl.BlockSpec((1, VD), lambda i: (0, 0)),
        ],
        out_specs=pl.BlockSpec((BN, VD), lambda i: (i, 0)),
        out_shape=jax.ShapeDtypeStruct((n, VD), jnp.float32),
        compiler_params=pltpu.CompilerParams(
            dimension_semantics=("parallel",),
        ),
        name="mhm_main",
    )(k, mkt, g, seg, bf.reshape(1, VD))
    return out


# speedup vs baseline: 1.6929x; 1.0367x over previous
import jax
import jax.numpy as jnp
from jax.experimental import pallas as pl
from jax.experimental.pallas import tpu as pltpu

H, M, D, KD, VD = 8, 64, 128, 128, 128
BN = 16384


def _prep_kernel(mems_ref, wk_ref, bk_ref, wv_ref, wf_ref, bv_ref,
                 mkt_ref, g_ref):
    hm2 = (H * M) // 2
    for h in range(H):
        mems_h = mems_ref[h]
        logits = jnp.dot(mems_h, wk_ref[h], preferred_element_type=jnp.float32)
        logits = logits + bk_ref[h]
        mx = jnp.max(logits, axis=1, keepdims=True)
        e = jnp.exp(logits - mx)
        mk = e / jnp.sum(e, axis=1, keepdims=True)
        mkt = mk.T
        c = h * (M // 2)
        mkt_ref[:, c:c + M // 2] = mkt[:, :M // 2]
        mkt_ref[:, hm2 + c:hm2 + c + M // 2] = mkt[:, M // 2:]
        mem_val = jnp.dot(mems_h, wv_ref[h], preferred_element_type=jnp.float32)
        mem_val = mem_val + bv_ref[h]
        gh = jnp.dot(mem_val, wf_ref[h * VD:(h + 1) * VD, :],
                     preferred_element_type=jnp.float32)
        g_ref[c:c + M // 2, :] = gh[:M // 2, :]
        g_ref[hm2 + c:hm2 + c + M // 2, :] = gh[M // 2:, :]


CB = 1024


def _main_kernel(k_ref, mkt_ref, g_ref, seg_ref, bf_ref, o_ref):
    half = (H * M) // 2
    for c in range(BN // CB):
        rows = slice(c * CB, (c + 1) * CB)
        kb = k_ref[rows, :]
        att = jnp.dot(kb, mkt_ref[...], preferred_element_type=jnp.float32)
        e = jnp.exp(att)
        f = e[:, :half] + e[:, half:]
        s = jnp.dot(f, seg_ref[...], preferred_element_type=jnp.float32)
        r = 1.0 / s
        att_w = e * jnp.concatenate([r, r], axis=1)
        out = jax.lax.dot_general(
            att_w[:, :half], g_ref[:half, :],
            dimension_numbers=(((1,), (0,)), ((), ())),
            preferred_element_type=jnp.float32)
        out = out + jax.lax.dot_general(
            att_w[:, half:], g_ref[half:, :],
            dimension_numbers=(((1,), (0,)), ((), ())),
            preferred_element_type=jnp.float32)
        o_ref[rows, :] = out + bf_ref[...]


@jax.jit
def kernel(k, mems, Wk, bk, Wv, bv, Wf, bf):
    n = k.shape[0]
    mkt, g = pl.pallas_call(
        _prep_kernel,
        out_shape=[
            jax.ShapeDtypeStruct((KD, H * M), jnp.float32),
            jax.ShapeDtypeStruct((H * M, VD), jnp.float32),
        ],
        name="mhm_prep",
    )(mems, Wk, bk.reshape(H, 1, KD), Wv, Wf, bv.reshape(H, 1, VD))

    hm = H * M
    head_of_row = jnp.arange(hm // 2) // (M // 2)
    head_of_col = jnp.arange(hm // 2) // (M // 2)
    seg = (head_of_row[:, None] == head_of_col[None, :]).astype(jnp.float32)

    out = pl.pallas_call(
        _main_kernel,
        grid=(n // BN,),
        in_specs=[
            pl.BlockSpec((BN, KD), lambda i: (i, 0)),
            pl.BlockSpec((KD, hm), lambda i: (0, 0)),
            pl.BlockSpec((hm, VD), lambda i: (0, 0)),
            pl.BlockSpec((hm // 2, hm // 2), lambda i: (0, 0)),
            pl.BlockSpec((1, VD), lambda i: (0, 0)),
        ],
        out_specs=pl.BlockSpec((BN, VD), lambda i: (i, 0)),
        out_shape=jax.ShapeDtypeStruct((n, VD), jnp.float32),
        compiler_params=pltpu.CompilerParams(
            dimension_semantics=("parallel",),
        ),
        name="mhm_main",
    )(k, mkt, g, seg, bf.reshape(1, VD))
    return out


# speedup vs baseline: 1.7031x; 1.0060x over previous
import jax
import jax.numpy as jnp
from jax.experimental import pallas as pl
from jax.experimental.pallas import tpu as pltpu

H, M, D, KD, VD = 8, 64, 128, 128, 128
BN = 16384


def _prep_kernel(mems_ref, wk_ref, bk_ref, wv_ref, wf_ref, bv_ref,
                 mkt_ref, g_ref):
    hm2 = (H * M) // 2
    for h in range(H):
        mems_h = mems_ref[h]
        logits = jnp.dot(mems_h, wk_ref[h], preferred_element_type=jnp.float32)
        logits = logits + bk_ref[h]
        mx = jnp.max(logits, axis=1, keepdims=True)
        e = jnp.exp(logits - mx)
        mk = e / jnp.sum(e, axis=1, keepdims=True)
        mkt = mk.T * 1.4426950408889634
        c = h * (M // 2)
        mkt_ref[:, c:c + M // 2] = mkt[:, :M // 2]
        mkt_ref[:, hm2 + c:hm2 + c + M // 2] = mkt[:, M // 2:]
        mem_val = jnp.dot(mems_h, wv_ref[h], preferred_element_type=jnp.float32)
        mem_val = mem_val + bv_ref[h]
        gh = jnp.dot(mem_val, wf_ref[h * VD:(h + 1) * VD, :],
                     preferred_element_type=jnp.float32)
        g_ref[c:c + M // 2, :] = gh[:M // 2, :]
        g_ref[hm2 + c:hm2 + c + M // 2, :] = gh[M // 2:, :]


CB = 1024


def _main_kernel(k_ref, mkt_ref, g_ref, seg_ref, bf_ref, o_ref):
    half = (H * M) // 2
    for c in range(BN // CB):
        rows = slice(c * CB, (c + 1) * CB)
        kb = k_ref[rows, :]
        att = jnp.dot(kb, mkt_ref[...], preferred_element_type=jnp.float32)
        e = jnp.exp2(att)
        f = e[:, :half] + e[:, half:]
        s = jnp.dot(f, seg_ref[...], preferred_element_type=jnp.float32)
        r = 1.0 / s
        att_w = e * jnp.concatenate([r, r], axis=1)
        out = jax.lax.dot_general(
            att_w[:, :half], g_ref[:half, :],
            dimension_numbers=(((1,), (0,)), ((), ())),
            preferred_element_type=jnp.float32)
        out = out + jax.lax.dot_general(
            att_w[:, half:], g_ref[half:, :],
            dimension_numbers=(((1,), (0,)), ((), ())),
            preferred_element_type=jnp.float32)
        o_ref[rows, :] = out + bf_ref[...]


@jax.jit
def kernel(k, mems, Wk, bk, Wv, bv, Wf, bf):
    n = k.shape[0]
    mkt, g = pl.pallas_call(
        _prep_kernel,
        out_shape=[
            jax.ShapeDtypeStruct((KD, H * M), jnp.float32),
            jax.ShapeDtypeStruct((H * M, VD), jnp.float32),
        ],
        name="mhm_prep",
    )(mems, Wk, bk.reshape(H, 1, KD), Wv, Wf, bv.reshape(H, 1, VD))

    hm = H * M
    head_of_row = jnp.arange(hm // 2) // (M // 2)
    head_of_col = jnp.arange(hm // 2) // (M // 2)
    seg = (head_of_row[:, None] == head_of_col[None, :]).astype(jnp.float32)

    out = pl.pallas_call(
        _main_kernel,
        grid=(n // BN,),
        in_specs=[
            pl.BlockSpec((BN, KD), lambda i: (i, 0)),
            pl.BlockSpec((KD, hm), lambda i: (0, 0)),
            pl.BlockSpec((hm, VD), lambda i: (0, 0)),
            pl.BlockSpec((hm // 2, hm // 2), lambda i: (0, 0)),
            pl.BlockSpec((1, VD), lambda i: (0, 0)),
        ],
        out_specs=pl.BlockSpec((BN, VD), lambda i: (i, 0)),
        out_shape=jax.ShapeDtypeStruct((n, VD), jnp.float32),
        compiler_params=pltpu.CompilerParams(
            dimension_semantics=("parallel",),
        ),
        name="mhm_main",
    )(k, mkt, g, seg, bf.reshape(1, VD))
    return out
